# Initial kernel scaffold; baseline (speedup 1.0000x reference)
import jax
import jax.numpy as jnp
from jax.experimental import pallas as pl


def kernel(x, edge_index, batch, W_gcn_0, b_gcn_0, W_gcn_h1, b_gcn_h1, W_gcn_h2, b_gcn_h2, W_gcn_h3, b_gcn_h3, W_gcn_h4, b_gcn_h4, W_gcn_h5, b_gcn_h5, W_gcn_h6, b_gcn_h6, W_gcn_out, b_gcn_out):
    raise NotImplementedError("write your pallas kernel here")


# speedup vs baseline: 11.4370x; 11.4370x over previous
import functools

import jax
import jax.numpy as jnp
from jax import lax
from jax.experimental import pallas as pl
from jax.experimental.pallas import tpu as pltpu
from jax.experimental.pallas import tpu_sc as plsc

N_NODES = 10000
NP = 10112
RPT = NP // 16
E = 320000
NW = 32
CHUNK = 128
NCHUNK = 79
EPT = CHUNK * NCHUNK
EPAD = EPT * NW

_MESH = plsc.VectorSubcoreMesh(core_axis_name="c", subcore_axis_name="s")


def _make_propagate(D):
    @functools.partial(
        pl.kernel,
        out_type=jax.ShapeDtypeStruct((2, NP, D), jnp.float32),
        mesh=_MESH,
        scratch_types=[
            pltpu.VMEM((CHUNK,), jnp.int32),
            pltpu.VMEM((CHUNK,), jnp.int32),
            pltpu.VMEM((CHUNK, D), jnp.float32),
            pltpu.VMEM_SHARED((NP, D), jnp.float32),
            pltpu.SemaphoreType.DMA,
        ],
        compiler_params=pltpu.CompilerParams(use_tc_tiling_on_sc=False),
    )
    def prop(zs_hbm, src_hbm, dst_hbm, zeros_hbm, out_hbm,
             src_v, dst_v, rows_v, acc_sh, sem):
        c = lax.axis_index("c")
        s = lax.axis_index("s")
        wid = c * 16 + s
        base = s * RPT
        pltpu.sync_copy(zeros_hbm.at[pl.ds(base, RPT)],
                        acc_sh.at[pl.ds(base, RPT)])
        plsc.subcore_barrier()

        def body(ci, carry):
            off = ci * CHUNK
            pltpu.sync_copy(src_hbm.at[wid, pl.ds(off, CHUNK)], src_v)
            pltpu.sync_copy(dst_hbm.at[wid, pl.ds(off, CHUNK)], dst_v)
            pltpu.async_copy(zs_hbm.at[src_v], rows_v, sem).wait()
            pltpu.sync_copy(rows_v, acc_sh.at[dst_v], add=True)
            return carry

        lax.fori_loop(0, NCHUNK, body, 0)
        plsc.subcore_barrier()
        pltpu.sync_copy(acc_sh.at[pl.ds(base, RPT)],
                        out_hbm.at[c, pl.ds(base, RPT)])

    return prop


_prop64 = _make_propagate(64)
_prop16 = _make_propagate(16)


@functools.partial(
    pl.kernel,
    out_type=jax.ShapeDtypeStruct((2, NP, 16), jnp.float32),
    mesh=_MESH,
    scratch_types=[
        pltpu.VMEM((CHUNK,), jnp.int32),
        pltpu.VMEM((CHUNK, 16), jnp.float32),
        pltpu.VMEM_SHARED((NP, 16), jnp.float32),
    ],
    compiler_params=pltpu.CompilerParams(use_tc_tiling_on_sc=False),
)
def _degree(dst_hbm, ones_hbm, zeros_hbm, out_hbm, dst_v, ones_v, acc_sh):
    c = lax.axis_index("c")
    s = lax.axis_index("s")
    wid = c * 16 + s
    base = s * RPT
    pltpu.sync_copy(ones_hbm, ones_v)
    pltpu.sync_copy(zeros_hbm.at[pl.ds(base, RPT)],
                    acc_sh.at[pl.ds(base, RPT)])
    plsc.subcore_barrier()

    def body(ci, carry):
        off = ci * CHUNK
        pltpu.sync_copy(dst_hbm.at[wid, pl.ds(off, CHUNK)], dst_v)
        pltpu.sync_copy(ones_v, acc_sh.at[dst_v], add=True)
        return carry

    lax.fori_loop(0, NCHUNK, body, 0)
    plsc.subcore_barrier()
    pltpu.sync_copy(acc_sh.at[pl.ds(base, RPT)],
                    out_hbm.at[c, pl.ds(base, RPT)])


def _tc_prologue(deg, x_pad, w0):
    def body(deg_ref, x_ref, w_ref, dis_ref, zs_ref):
        d = deg_ref[0, :, 0:1] + deg_ref[1, :, 0:1] + 1.0
        dis = lax.rsqrt(d)
        dis_ref[...] = dis
        zs_ref[...] = jnp.dot(x_ref[...], w_ref[...],
                              preferred_element_type=jnp.float32) * dis

    return pl.pallas_call(
        body,
        out_shape=[jax.ShapeDtypeStruct((NP, 1), jnp.float32),
                   jax.ShapeDtypeStruct((NP, 64), jnp.float32)],
    )(deg, x_pad, w0)


def _tc_layer(p, zs, dis, b, w_next, d_next):
    def body(p_ref, zs_ref, dis_ref, b_ref, w_ref, out_ref):
        dis = dis_ref[...]
        t = (p_ref[0] + p_ref[1] + zs_ref[...]) * dis + b_ref[...]
        h = jnp.maximum(t, 0.0)
        out_ref[...] = jnp.dot(h, w_ref[...],
                               preferred_element_type=jnp.float32) * dis

    return pl.pallas_call(
        body,
        out_shape=jax.ShapeDtypeStruct((NP, d_next), jnp.float32),
    )(p, zs, dis, b, w_next)


def _tc_final(p, zs, dis, b):
    def body(p_ref, zs_ref, dis_ref, b_ref, out_ref):
        t = (p_ref[0] + p_ref[1] + zs_ref[...]) * dis_ref[...] + b_ref[...]
        out_ref[...] = jax.nn.sigmoid(t)

    return pl.pallas_call(
        body,
        out_shape=jax.ShapeDtypeStruct((NP, 16), jnp.float32),
    )(p, zs, dis, b)


def kernel(x, edge_index, batch,
           W_gcn_0, b_gcn_0, W_gcn_h1, b_gcn_h1, W_gcn_h2, b_gcn_h2,
           W_gcn_h3, b_gcn_h3, W_gcn_h4, b_gcn_h4, W_gcn_h5, b_gcn_h5,
           W_gcn_h6, b_gcn_h6, W_gcn_out, b_gcn_out):
    del batch
    src = edge_index[0].astype(jnp.int32)
    dst = edge_index[1].astype(jnp.int32)
    src2 = jnp.pad(src, (0, EPAD - E)).reshape(NW, EPT)
    dst2 = jnp.pad(dst, (0, EPAD - E),
                   constant_values=NP - 1).reshape(NW, EPT)
    x_pad = jnp.pad(x, ((0, NP - N_NODES), (0, 0)))

    z64 = jnp.zeros((NP, 64), jnp.float32)
    z16 = jnp.zeros((NP, 16), jnp.float32)
    ones_c = jnp.ones((CHUNK, 16), jnp.float32)

    deg = _degree(dst2, ones_c, z16)
    dis, zs = _tc_prologue(deg, x_pad, W_gcn_0)

    ws = [W_gcn_h1, W_gcn_h2, W_gcn_h3, W_gcn_h4, W_gcn_h5, W_gcn_h6]
    bs = [b_gcn_0, b_gcn_h1, b_gcn_h2, b_gcn_h3, b_gcn_h4, b_gcn_h5,
          b_gcn_h6]
    w_out16 = jnp.pad(W_gcn_out, ((0, 0), (0, 10)))
    b_out16 = jnp.pad(b_gcn_out, (0, 10)).reshape(1, 16)

    for i in range(7):
        p = _prop64(zs, src2, dst2, z64)
        if i < 6:
            zs = _tc_layer(p, zs, dis, bs[i].reshape(1, 64), ws[i], 64)
        else:
            zs = _tc_layer(p, zs, dis, bs[i].reshape(1, 64), w_out16, 16)

    p = _prop16(zs, src2, dst2, z16)
    out = _tc_final(p, zs, dis, b_out16)
    return out[:N_NODES, :6]
